# baseline (speedup 1.0000x reference)
import functools

import jax
import jax.numpy as jnp
from jax import lax
from jax.experimental import pallas as pl
from jax.experimental.pallas import tpu as pltpu
from jax.experimental.pallas import tpu_sc as plsc

D = 64
S = 128
B = 2
NF = 2
HEADS = 8
REPS = HEADS // NF
VOCAB = 34

NC, NS, L = 2, 16, 16
NW = NC * NS

IDX_ROWS = NF * B * S
ROWS_PER_TILE = IDX_ROWS // NW
CHUNK = 256
N_CHUNK = ROWS_PER_TILE * S // CHUNK
GATHERS_PER_CHUNK = CHUNK // S
SLAB_SZ = S * S
OUT_ROWS = B * HEADS * SLAB_SZ


def _body(idx_hbm, tk_hbm, tv_hbm, outk_hbm, outv_hbm,
          idx_v, k_rows, v_rows, gsem, wsem0, wsem1):
    wid = lax.axis_index("s") * NC + lax.axis_index("c")
    slab = wid // (NW // (NF * B))
    part = wid % (NW // (NF * B))
    f = slab // B
    b = slab % B

    pltpu.sync_copy(idx_hbm.at[pl.ds(wid * ROWS_PER_TILE, ROWS_PER_TILE)],
                    idx_v)

    off = f * VOCAB
    for r in range(ROWS_PER_TILE):
        for c in range(S // L):
            sl = (r, pl.ds(c * L, L))
            idx_v[sl] = idx_v[sl] + off

    wsems = (wsem0, wsem1)
    pending = [[], []]

    for ci in range(N_CHUNK):
        sel = ci % 2
        bsl = pl.ds(sel * CHUNK, CHUNK)

        for c in pending[sel]:
            c.wait()
        pending[sel] = []

        copies = []
        for j in range(GATHERS_PER_CHUNK):
            row = ci * GATHERS_PER_CHUNK + j
            dst = pl.ds(sel * CHUNK + j * S, S)
            copies.append(pltpu.async_copy(
                tk_hbm.at[idx_v.at[row]], k_rows.at[dst], gsem))
            copies.append(pltpu.async_copy(
                tv_hbm.at[idx_v.at[row]], v_rows.at[dst], gsem))
        for c in copies:
            c.wait()

        base = part * (N_CHUNK * CHUNK) + ci * CHUNK
        for rep in range(REPS):
            head_row = (b * HEADS + f * REPS + rep) * SLAB_SZ + base
            pending[sel].append(pltpu.async_copy(
                k_rows.at[bsl], outk_hbm.at[pl.ds(head_row, CHUNK)],
                wsems[sel]))
            pending[sel].append(pltpu.async_copy(
                v_rows.at[bsl], outv_hbm.at[pl.ds(head_row, CHUNK)],
                wsems[sel]))

    for sel in range(2):
        for c in pending[sel]:
            c.wait()


@jax.jit
def _sc_lookup(idx, t_k, t_v):
    run = pl.kernel(
        _body,
        out_type=[jax.ShapeDtypeStruct((OUT_ROWS, D), jnp.float32),
                  jax.ShapeDtypeStruct((OUT_ROWS, D), jnp.float32)],
        mesh=plsc.VectorSubcoreMesh(core_axis_name="c", subcore_axis_name="s",
                                    num_cores=NC, num_subcores=NS),
        scratch_types=[
            pltpu.VMEM((ROWS_PER_TILE, S), jnp.int32),
            pltpu.VMEM((2 * CHUNK, D), jnp.float32),
            pltpu.VMEM((2 * CHUNK, D), jnp.float32),
            pltpu.SemaphoreType.DMA,
            pltpu.SemaphoreType.DMA,
            pltpu.SemaphoreType.DMA,
        ],
        compiler_params=pltpu.CompilerParams(use_tc_tiling_on_sc=False),
    )
    return run(idx, t_k, t_v)


def kernel(inputs, emb0, emb1):
    scale = jnp.float32(8.0)
    t_k = jnp.concatenate([emb0[:, :D], emb1[:, :D]], axis=0) * scale
    t_v = jnp.concatenate([emb0[:, D:], emb1[:, D:]], axis=0) * scale
    idx = inputs.reshape(IDX_ROWS, S)
    out_k, out_v = _sc_lookup(idx, t_k, t_v)
    return (out_k.reshape(B, HEADS, S, S, D),
            out_v.reshape(B, HEADS, S, S, D))
